# baseline (speedup 1.0000x reference)
import jax
import jax.numpy as jnp
from jax import lax
from jax.experimental import pallas as pl
from jax.experimental.pallas import tpu as pltpu
from jax.experimental.pallas import tpu_sc as plsc

_F32 = jnp.float32
_BF16 = jnp.bfloat16
_I32 = jnp.int32

_NC = 2
_NS = 16
_NW = _NC * _NS
_LANES = 16


def _round_bf16(v):
    u = lax.bitcast_convert_type(v, _I32)
    lsb = jnp.bitwise_and(jnp.right_shift(u, 16), 1)
    r = jnp.bitwise_and(u + 32767 + lsb, _I32(-65536))
    return lax.bitcast_convert_type(r, _F32)


def _make_esum(b_g, n, h_dim, chunk_rows):
    rows = b_g * n
    rows_per_w = rows // _NW
    n_chunks = rows_per_w // chunk_rows
    jvecs = n // _LANES
    assert h_dim == 4 * _LANES and rows_per_w * _NW == rows
    assert n % rows_per_w == 0
    assert n_chunks * chunk_rows == rows_per_w and n % _LANES == 0

    def body(e_hbm, out_hbm, buf0, buf1, outbuf, sem0, sem1):
        wid = lax.axis_index("s") * _NC + lax.axis_index("c")
        b = wid // (n // rows_per_w)
        i0 = (wid % (n // rows_per_w)) * rows_per_w
        bufs = (buf0, buf1)
        sems = (sem0, sem1)

        def start(c):
            return pltpu.async_copy(
                e_hbm.at[b, pl.ds(i0 + c * chunk_rows, chunk_rows)],
                bufs[c % 2], sems[c % 2])

        cp = start(0)
        for c in range(n_chunks):
            nxt = start(c + 1) if c + 1 < n_chunks else None
            cp.wait()
            buf = bufs[c % 2]
            for r in range(chunk_rows):
                orow = c * chunk_rows + r

                def hbody(h, carry, buf=buf, r=r, orow=orow):
                    p = [_round_bf16(buf[r, h, pl.ds(g * _LANES, _LANES)])
                         for g in range(jvecs)]
                    while len(p) > 1:
                        p = [a + b for a, b in zip(p[::2], p[1::2])]
                    outbuf[orow, pl.ds(h * _LANES, _LANES)] = p[0]
                    return carry

                lax.fori_loop(0, h_dim, hbody, 0)
            cp = nxt
        pltpu.sync_copy(
            outbuf, out_hbm.at[pl.ds(wid * rows_per_w, rows_per_w)])

    return pl.kernel(
        body,
        mesh=plsc.VectorSubcoreMesh(core_axis_name="c", subcore_axis_name="s"),
        out_type=jax.ShapeDtypeStruct((rows, h_dim * _LANES), _F32),
        scratch_types=[
            pltpu.VMEM((chunk_rows, h_dim, n), _F32),
            pltpu.VMEM((chunk_rows, h_dim, n), _F32),
            pltpu.VMEM((rows_per_w, h_dim * _LANES), _F32),
            pltpu.SemaphoreType.DMA,
            pltpu.SemaphoreType.DMA,
        ],
    )


def _dot(a, b, precision=None):
    return lax.dot_general(
        a, b, (((1,), (0,)), ((), ())),
        precision=precision,
        preferred_element_type=_F32,
    )


def _gru_kernel(y_ref, node_ref, wfab_ref, wfc_ref, bf_ref, wih_ref,
                whh_ref, bih_ref, bhh_ref, out_ref):
    b_g, n, h_dim = node_ref.shape
    rows = b_g * n
    r_rounds = wfab_ref.shape[0]
    hi = lax.Precision.HIGHEST

    yp = y_ref[...]
    y = jnp.sum(yp.reshape(rows, h_dim, _LANES), axis=2)
    h = node_ref[...].reshape(rows, h_dim)
    fn = _F32(n)
    for t in range(r_rounds):
        h16 = h.astype(_BF16)
        hterm = _dot(h16, wfab_ref[t, :h_dim, :]) * fn
        hs = jnp.sum(h16.astype(_F32).reshape(b_g, n, h_dim), axis=1)
        hsterm = _dot(hs, wfab_ref[t, h_dim:, :].astype(_F32), hi)
        hsrows = jnp.broadcast_to(hsterm[:, None, :], (b_g, n, 6 * h_dim))
        eterm = _dot(y, wfc_ref[t].astype(_F32), hi)
        agg = (hterm + hsrows.reshape(rows, 6 * h_dim) + eterm
               + fn * bf_ref[t][None, :])
        gi = _dot(agg.astype(_BF16), wih_ref[t]) + bih_ref[t][None, :]
        gh = _dot(h16, whh_ref[t]) + bhh_ref[t][None, :]
        i_r, i_z, i_n = jnp.split(gi, 3, axis=-1)
        h_r, h_z, h_n = jnp.split(gh, 3, axis=-1)
        r = jax.nn.sigmoid(i_r + h_r)
        z = jax.nn.sigmoid(i_z + h_z)
        nn = jnp.tanh(i_n + r * h_n)
        h = (1.0 - z) * nn + z * h

    out_ref[...] = jnp.sum(h.reshape(b_g, n, h_dim), axis=1)


def kernel(embedded_node, embedded_adjancy_matrix, Wf, bf, Wih, Whh, bih, bhh):
    b_g, n, _, h_dim = embedded_adjancy_matrix.shape
    r_rounds = Wf.shape[0]
    rows = b_g * n

    e_t = jnp.swapaxes(embedded_adjancy_matrix, 2, 3)
    y = _make_esum(b_g, n, h_dim, chunk_rows=2)(e_t)

    wfc16 = Wf[:, 2 * h_dim:, :].astype(_BF16)
    wfab16 = Wf[:, :2 * h_dim, :].astype(_BF16)
    wih16 = Wih.astype(_BF16)
    whh16 = Whh.astype(_BF16)

    out = pl.pallas_call(
        _gru_kernel,
        out_shape=jax.ShapeDtypeStruct((b_g, h_dim), jnp.float32),
    )(y, embedded_node, wfab16, wfc16, bf, wih16, whh16, bih, bhh)
    return out


# speedup vs baseline: 1.2507x; 1.2507x over previous
import jax
import jax.numpy as jnp
from jax import lax
from jax.experimental import pallas as pl
from jax.experimental.pallas import tpu as pltpu
from jax.experimental.pallas import tpu_sc as plsc

_F32 = jnp.float32
_BF16 = jnp.bfloat16
_I32 = jnp.int32

_NC = 2
_NS = 16
_NW = _NC * _NS
_LANES = 16


def _round_bf16(v):
    u = lax.bitcast_convert_type(v, _I32)
    lsb = jnp.bitwise_and(jnp.right_shift(u, 16), 1)
    r = jnp.bitwise_and(u + 32767 + lsb, _I32(-65536))
    return lax.bitcast_convert_type(r, _F32)


def _make_esum(b_g, n, h_dim, chunk_rows):
    rows = b_g * n
    rows_per_w = rows // _NW
    n_chunks = rows_per_w // chunk_rows
    jvecs = n // _LANES
    assert h_dim == 4 * _LANES and rows_per_w * _NW == rows
    assert n % rows_per_w == 0
    assert n_chunks * chunk_rows == rows_per_w and n % _LANES == 0

    def body(e_hbm, out_hbm, buf0, buf1, outbuf, sem0, sem1):
        wid = lax.axis_index("s") * _NC + lax.axis_index("c")
        b = wid // (n // rows_per_w)
        i0 = (wid % (n // rows_per_w)) * rows_per_w
        bufs = (buf0, buf1)
        sems = (sem0, sem1)

        def start(c):
            return pltpu.async_copy(
                e_hbm.at[b, pl.ds(i0 + c * chunk_rows, chunk_rows)],
                bufs[c % 2], sems[c % 2])

        cp = start(0)
        for c in range(n_chunks):
            nxt = start(c + 1) if c + 1 < n_chunks else None
            cp.wait()
            buf = bufs[c % 2]
            for r in range(chunk_rows):
                orow = c * chunk_rows + r

                def hbody(h, carry, buf=buf, r=r, orow=orow):
                    p = [_round_bf16(buf[r, h, pl.ds(g * _LANES, _LANES)])
                         for g in range(jvecs)]
                    while len(p) > 1:
                        p = [a + b for a, b in zip(p[::2], p[1::2])]
                    outbuf[orow, pl.ds(h * _LANES, _LANES)] = p[0]
                    return carry

                lax.fori_loop(0, h_dim, hbody, 0)
            cp = nxt
        pltpu.sync_copy(
            outbuf, out_hbm.at[pl.ds(wid * rows_per_w, rows_per_w)])

    return pl.kernel(
        body,
        mesh=plsc.VectorSubcoreMesh(core_axis_name="c", subcore_axis_name="s"),
        out_type=jax.ShapeDtypeStruct((rows, h_dim * _LANES), _F32),
        scratch_types=[
            pltpu.VMEM((chunk_rows, h_dim, n), _F32),
            pltpu.VMEM((chunk_rows, h_dim, n), _F32),
            pltpu.VMEM((rows_per_w, h_dim * _LANES), _F32),
            pltpu.SemaphoreType.DMA,
            pltpu.SemaphoreType.DMA,
        ],
    )


def _dot(a, b, precision=None):
    return lax.dot_general(
        a, b, (((1,), (0,)), ((), ())),
        precision=precision,
        preferred_element_type=_F32,
    )


def _tc_esum_kernel(e_ref, agg_ref, out_ref):
    nb, h_dim, jn = e_ref.shape[1], e_ref.shape[2], e_ref.shape[3]
    a = e_ref[0].reshape(nb, h_dim * jn).astype(_BF16)
    out_ref[0] = _dot(a, agg_ref[...])


def _gru_kernel(ysc_ref, ytc_ref, node_ref, wfab_ref, wfc_ref, bf_ref,
                wih_ref, whh_ref, bih_ref, bhh_ref, out_ref):
    b_g, n, h_dim = node_ref.shape
    rows = b_g * n
    r_rounds = wfab_ref.shape[0]
    hi = lax.Precision.HIGHEST

    yp = ysc_ref[...]
    pidx = lax.broadcasted_iota(jnp.int32, (yp.shape[1], h_dim), 0)
    hidx = lax.broadcasted_iota(jnp.int32, (yp.shape[1], h_dim), 1)
    sel = (pidx // _LANES == hidx).astype(_F32)
    ysc = _dot(yp, sel, lax.Precision.HIGHEST)
    ytc = ytc_ref[...].reshape(rows - ysc.shape[0], h_dim)
    y = jnp.concatenate([ysc, ytc], axis=0)
    h = node_ref[...].reshape(rows, h_dim)
    fn = _F32(n)
    for t in range(r_rounds):
        h16 = h.astype(_BF16)
        hterm = _dot(h16, wfab_ref[t, :h_dim, :]) * fn
        hs = jnp.sum(h16.astype(_F32).reshape(b_g, n, h_dim), axis=1)
        hsterm = _dot(hs, wfab_ref[t, h_dim:, :].astype(_F32), hi)
        hsrows = jnp.broadcast_to(hsterm[:, None, :], (b_g, n, 6 * h_dim))
        eterm = _dot(y, wfc_ref[t].astype(_F32), hi)
        agg = (hterm + hsrows.reshape(rows, 6 * h_dim) + eterm
               + fn * bf_ref[t][None, :])
        gi = _dot(agg.astype(_BF16), wih_ref[t]) + bih_ref[t][None, :]
        gh = _dot(h16, whh_ref[t]) + bhh_ref[t][None, :]
        i_r, i_z, i_n = jnp.split(gi, 3, axis=-1)
        h_r, h_z, h_n = jnp.split(gh, 3, axis=-1)
        r = jax.nn.sigmoid(i_r + h_r)
        z = jax.nn.sigmoid(i_z + h_z)
        nn = jnp.tanh(i_n + r * h_n)
        h = (1.0 - z) * nn + z * h

    out_ref[...] = jnp.sum(h.reshape(b_g, n, h_dim), axis=1)


def kernel(embedded_node, embedded_adjancy_matrix, Wf, bf, Wih, Whh, bih, bhh):
    b_g, n, _, h_dim = embedded_adjancy_matrix.shape
    r_rounds = Wf.shape[0]
    rows = b_g * n

    e_t = jnp.swapaxes(embedded_adjancy_matrix, 2, 3)
    b_sc = b_g // 2
    b_tc = b_g - b_sc
    y_sc = _make_esum(b_sc, n, h_dim, chunk_rows=2)(e_t)

    agg = (jnp.arange(h_dim * n)[:, None] // n
           == jnp.arange(h_dim)[None, :]).astype(_BF16)
    nb = 32
    y_tc = pl.pallas_call(
        _tc_esum_kernel,
        grid=(b_tc, n // nb),
        in_specs=[
            pl.BlockSpec((1, nb, h_dim, n), lambda b, i: (b_sc + b, i, 0, 0)),
            pl.BlockSpec((h_dim * n, h_dim), lambda b, i: (0, 0)),
        ],
        out_specs=pl.BlockSpec((1, nb, h_dim), lambda b, i: (b, i, 0)),
        out_shape=jax.ShapeDtypeStruct((b_tc, n, h_dim), jnp.float32),
        compiler_params=pltpu.CompilerParams(
            dimension_semantics=("arbitrary", "arbitrary"),
        ),
    )(e_t, agg)

    wfc16 = Wf[:, 2 * h_dim:, :].astype(_BF16)
    wfab16 = Wf[:, :2 * h_dim, :].astype(_BF16)
    wih16 = Wih.astype(_BF16)
    whh16 = Whh.astype(_BF16)

    out = pl.pallas_call(
        _gru_kernel,
        out_shape=jax.ShapeDtypeStruct((b_g, h_dim), jnp.float32),
    )(y_sc, y_tc, embedded_node, wfab16, wfc16, bf, wih16, whh16, bih, bhh)
    return out


# speedup vs baseline: 1.3699x; 1.0953x over previous
import jax
import jax.numpy as jnp
import numpy as np
from jax import lax
from jax.experimental import pallas as pl
from jax.experimental.pallas import tpu as pltpu
from jax.experimental.pallas import tpu_sc as plsc

_F32 = jnp.float32
_BF16 = jnp.bfloat16
_I32 = jnp.int32

_NC = 2
_NS = 16
_NW = _NC * _NS
_LANES = 16


def _round_bf16(v):
    u = lax.bitcast_convert_type(v, _I32)
    lsb = jnp.bitwise_and(jnp.right_shift(u, 16), 1)
    r = jnp.bitwise_and(u + 32767 + lsb, _I32(-65536))
    return lax.bitcast_convert_type(r, _F32)


def _make_esum(b_g, n, h_dim, chunk_rows):
    rows = b_g * n
    rows_per_w = rows // _NW
    n_chunks = rows_per_w // chunk_rows
    jvecs = n // _LANES
    assert h_dim == 4 * _LANES and rows_per_w * _NW == rows
    assert n % rows_per_w == 0
    assert n_chunks * chunk_rows == rows_per_w and n % _LANES == 0

    def body(e_hbm, out_hbm, buf0, buf1, outbuf, sem0, sem1):
        wid = lax.axis_index("s") * _NC + lax.axis_index("c")
        b = wid // (n // rows_per_w)
        i0 = (wid % (n // rows_per_w)) * rows_per_w
        bufs = (buf0, buf1)
        sems = (sem0, sem1)

        def start(c):
            return pltpu.async_copy(
                e_hbm.at[b, pl.ds(i0 + c * chunk_rows, chunk_rows)],
                bufs[c % 2], sems[c % 2])

        cp = start(0)
        for c in range(n_chunks):
            nxt = start(c + 1) if c + 1 < n_chunks else None
            cp.wait()
            buf = bufs[c % 2]
            for r in range(chunk_rows):
                orow = c * chunk_rows + r

                def hbody(h, carry, buf=buf, r=r, orow=orow):
                    p = [_round_bf16(buf[r, h, pl.ds(g * _LANES, _LANES)])
                         for g in range(jvecs)]
                    while len(p) > 1:
                        p = [a + b for a, b in zip(p[::2], p[1::2])]
                    outbuf[orow, pl.ds(h * _LANES, _LANES)] = p[0]
                    return carry

                lax.fori_loop(0, h_dim, hbody, 0)
            cp = nxt
        pltpu.sync_copy(
            outbuf, out_hbm.at[pl.ds(wid * rows_per_w, rows_per_w)])

    return pl.kernel(
        body,
        mesh=plsc.VectorSubcoreMesh(core_axis_name="c", subcore_axis_name="s"),
        out_type=jax.ShapeDtypeStruct((rows, h_dim * _LANES), _F32),
        scratch_types=[
            pltpu.VMEM((chunk_rows, h_dim, n), _F32),
            pltpu.VMEM((chunk_rows, h_dim, n), _F32),
            pltpu.VMEM((rows_per_w, h_dim * _LANES), _F32),
            pltpu.SemaphoreType.DMA,
            pltpu.SemaphoreType.DMA,
        ],
    )


def _dot(a, b, precision=None):
    return lax.dot_general(
        a, b, (((1,), (0,)), ((), ())),
        precision=precision,
        preferred_element_type=_F32,
    )


def _tc_esum_kernel(e_ref, agg_ref, out_ref):
    nb, h_dim, jn = e_ref.shape[1], e_ref.shape[2], e_ref.shape[3]
    a = e_ref[0].reshape(nb, h_dim * jn).astype(_BF16)
    out_ref[0] = _dot(a, agg_ref[...])


def _gru_kernel(ysc_ref, ytc_ref, node_ref, wfab_ref, wfc_ref, bf_ref,
                wih_ref, whh_ref, bih_ref, bhh_ref, out_ref):
    b_g, n, h_dim = node_ref.shape
    rows = b_g * n
    r_rounds = wfab_ref.shape[0]
    hi = lax.Precision.HIGHEST

    yp = ysc_ref[...]
    pidx = lax.broadcasted_iota(jnp.int32, (yp.shape[1], h_dim), 0)
    hidx = lax.broadcasted_iota(jnp.int32, (yp.shape[1], h_dim), 1)
    sel = (pidx // _LANES == hidx).astype(_F32)
    ysc = _dot(yp, sel, lax.Precision.HIGHEST)
    ytc = ytc_ref[...].reshape(rows - ysc.shape[0], h_dim)
    y = jnp.concatenate([ysc, ytc], axis=0)
    h = node_ref[...].reshape(rows, h_dim)
    fn = _F32(n)
    for t in range(r_rounds):
        h16 = h.astype(_BF16)
        hterm = _dot(h16, wfab_ref[t, :h_dim, :]) * fn
        hs = jnp.sum(h16.astype(_F32).reshape(b_g, n, h_dim), axis=1)
        hsterm = _dot(hs, wfab_ref[t, h_dim:, :].astype(_F32), hi)
        hsrows = jnp.broadcast_to(hsterm[:, None, :], (b_g, n, 6 * h_dim))
        eterm = _dot(y, wfc_ref[t].astype(_F32), hi)
        agg = (hterm + hsrows.reshape(rows, 6 * h_dim) + eterm
               + fn * bf_ref[t][None, :])
        gi = _dot(agg.astype(_BF16), wih_ref[t]) + bih_ref[t][None, :]
        gh = _dot(h16, whh_ref[t]) + bhh_ref[t][None, :]
        i_r, i_z, i_n = jnp.split(gi, 3, axis=-1)
        h_r, h_z, h_n = jnp.split(gh, 3, axis=-1)
        r = jax.nn.sigmoid(i_r + h_r)
        z = jax.nn.sigmoid(i_z + h_z)
        nn = jnp.tanh(i_n + r * h_n)
        h = (1.0 - z) * nn + z * h

    out_ref[...] = jnp.sum(h.reshape(b_g, n, h_dim), axis=1)


def kernel(embedded_node, embedded_adjancy_matrix, Wf, bf, Wih, Whh, bih, bhh):
    b_g, n, _, h_dim = embedded_adjancy_matrix.shape
    r_rounds = Wf.shape[0]
    rows = b_g * n

    e_t = jnp.swapaxes(embedded_adjancy_matrix, 2, 3)
    b_sc = b_g // 2
    b_tc = b_g - b_sc
    y_sc = _make_esum(b_sc, n, h_dim, chunk_rows=2)(e_t)

    agg = jnp.asarray((np.arange(h_dim * n)[:, None] // n
                       == np.arange(h_dim)[None, :]), dtype=_BF16)
    nb = 32
    y_tc = pl.pallas_call(
        _tc_esum_kernel,
        grid=(b_tc, n // nb),
        in_specs=[
            pl.BlockSpec((1, nb, h_dim, n), lambda b, i: (b_sc + b, i, 0, 0)),
            pl.BlockSpec((h_dim * n, h_dim), lambda b, i: (0, 0)),
        ],
        out_specs=pl.BlockSpec((1, nb, h_dim), lambda b, i: (b, i, 0)),
        out_shape=jax.ShapeDtypeStruct((b_tc, n, h_dim), jnp.float32),
        compiler_params=pltpu.CompilerParams(
            dimension_semantics=("arbitrary", "arbitrary"),
        ),
    )(e_t, agg)

    wfc16 = Wf[:, 2 * h_dim:, :].astype(_BF16)
    wfab16 = Wf[:, :2 * h_dim, :].astype(_BF16)
    wih16 = Wih.astype(_BF16)
    whh16 = Whh.astype(_BF16)

    out = pl.pallas_call(
        _gru_kernel,
        out_shape=jax.ShapeDtypeStruct((b_g, h_dim), jnp.float32),
    )(y_sc, y_tc, embedded_node, wfab16, wfc16, bf, wih16, whh16, bih, bhh)
    return out
